# baseline (speedup 1.0000x reference)
import functools

import jax
import jax.numpy as jnp
from jax import lax
from jax.experimental import pallas as pl
from jax.experimental.pallas import tpu as pltpu

_LANE = 128
_SUBLANE = 8


def _ceil_to(x, m):
    return (x + m - 1) // m * m


def _rnn_kernel(x_ref,
                w_ih_ref,
                w_hh_ref,
                b_rnn_ref,
                w_fc_ref,
                b_fc_ref,
                out_ref,
                xw_ref,
                *, unroll):
    T, Bt, E = x_ref.shape
    H = w_hh_ref.shape[0]

    xw = jnp.dot(x_ref[...].reshape(T * Bt, E), w_ih_ref[...],
                 preferred_element_type=jnp.float32)
    xw_ref[...] = (xw + b_rnn_ref[...]).astype(xw_ref.dtype).reshape(T, Bt, H)

    def step(t, h):
        pre = xw_ref[t].astype(jnp.float32) + jnp.dot(
            h, w_hh_ref[...], preferred_element_type=jnp.float32)
        return jnp.tanh(pre).astype(h.dtype)

    h = lax.fori_loop(0, T, step, jnp.zeros((Bt, H), jnp.bfloat16),
                      unroll=unroll)

    out_ref[...] = (jnp.dot(h, w_fc_ref[...],
                            preferred_element_type=jnp.float32)
                    + b_fc_ref[...]).astype(out_ref.dtype)


def kernel(x_tokens, embedding, w_ih, w_hh, b_ih, b_hh, w_fc, b_fc):
    B, T = x_tokens.shape
    E = embedding.shape[1]
    H = w_hh.shape[0]
    C = w_fc.shape[1]

    cdt = jnp.bfloat16

    Ep, Hp, Cp = (_ceil_to(d, _LANE) for d in (E, H, C))
    Bt = min(128, _ceil_to(B, _SUBLANE))
    Bp = _ceil_to(B, Bt)
    num_tiles = Bp // Bt

    x = jnp.take(embedding, x_tokens.T, axis=0)
    if (Bp, Ep) != (B, E):
        x = jnp.pad(x, ((0, 0), (0, Bp - B), (0, Ep - E)))

    def padc(a, r, c):
        out = jnp.pad(a, ((0, r - a.shape[0]), (0, c - a.shape[1])))
        return out

    w_ih_c = padc(w_ih, Ep, Hp)
    w_hh_c = padc(w_hh, Hp, Hp).astype(cdt)
    w_fc_c = padc(w_fc, Hp, Cp).astype(cdt)
    b_rnn = padc(b_ih + b_hh, 1, Hp)
    b_fc_p = padc(b_fc, 1, Cp)

    const = lambda i: (0, 0)
    out_padded = pl.pallas_call(
        functools.partial(_rnn_kernel, unroll=8),
        out_shape=jax.ShapeDtypeStruct((Bp, Cp), jnp.float32),
        grid=(num_tiles,),
        in_specs=[
            pl.BlockSpec((T, Bt, Ep), lambda i: (0, i, 0)),
            pl.BlockSpec((Ep, Hp), const),
            pl.BlockSpec((Hp, Hp), const),
            pl.BlockSpec((1, Hp), const),
            pl.BlockSpec((Hp, Cp), const),
            pl.BlockSpec((1, Cp), const),
        ],
        out_specs=pl.BlockSpec((Bt, Cp), lambda i: (i, 0)),
        scratch_shapes=[pltpu.VMEM((T, Bt, Hp), cdt)],
        compiler_params=pltpu.CompilerParams(
            dimension_semantics=("parallel",),
        ),
    )(x, w_ih_c, w_hh_c, b_rnn, w_fc_c, b_fc_p)

    if (Bp, Cp) != (B, C):
        out_padded = out_padded[:B, :C]
    return out_padded


# speedup vs baseline: 1.1507x; 1.1507x over previous
import functools

import jax
import jax.numpy as jnp
from jax import lax
from jax.experimental import pallas as pl
from jax.experimental.pallas import tpu as pltpu

_LANE = 128
_SUBLANE = 8


def _ceil_to(x, m):
    return (x + m - 1) // m * m


def _rnn_kernel(x_ref,
                w_ih_ref,
                w_hh_ref,
                b_rnn_ref,
                w_fc_ref,
                b_fc_ref,
                out_ref,
                xw_ref,
                *, unroll):
    T, Bt, E = x_ref.shape
    H = w_hh_ref.shape[0]

    xw = jnp.dot(x_ref[...].reshape(T * Bt, E), w_ih_ref[...],
                 preferred_element_type=jnp.float32)
    xw_ref[...] = (xw + b_rnn_ref[...]).astype(xw_ref.dtype).reshape(T, Bt, H)

    def step(t, h):
        pre = xw_ref[t].astype(jnp.float32) + jnp.dot(
            h, w_hh_ref[...], preferred_element_type=jnp.float32)
        return jnp.tanh(pre).astype(h.dtype)

    h = lax.fori_loop(0, T, step, jnp.zeros((Bt, H), jnp.bfloat16),
                      unroll=unroll)

    out_ref[...] = (jnp.dot(h, w_fc_ref[...],
                            preferred_element_type=jnp.float32)
                    + b_fc_ref[...]).astype(out_ref.dtype)


def kernel(x_tokens, embedding, w_ih, w_hh, b_ih, b_hh, w_fc, b_fc):
    B, T = x_tokens.shape
    E = embedding.shape[1]
    H = w_hh.shape[0]
    C = w_fc.shape[1]

    cdt = jnp.bfloat16

    Ep, Hp, Cp = (_ceil_to(d, _LANE) for d in (E, H, C))
    Bt = min(256, _ceil_to(B, _SUBLANE))
    Bp = _ceil_to(B, Bt)
    num_tiles = Bp // Bt

    x = jnp.take(embedding, x_tokens.T, axis=0)
    if (Bp, Ep) != (B, E):
        x = jnp.pad(x, ((0, 0), (0, Bp - B), (0, Ep - E)))

    def padc(a, r, c):
        out = jnp.pad(a, ((0, r - a.shape[0]), (0, c - a.shape[1])))
        return out

    w_ih_c = padc(w_ih, Ep, Hp)
    w_hh_c = padc(w_hh, Hp, Hp).astype(cdt)
    w_fc_c = padc(w_fc, Hp, Cp).astype(cdt)
    b_rnn = padc(b_ih + b_hh, 1, Hp)
    b_fc_p = padc(b_fc, 1, Cp)

    const = lambda i: (0, 0)
    out_padded = pl.pallas_call(
        functools.partial(_rnn_kernel, unroll=8),
        out_shape=jax.ShapeDtypeStruct((Bp, Cp), jnp.float32),
        grid=(num_tiles,),
        in_specs=[
            pl.BlockSpec((T, Bt, Ep), lambda i: (0, i, 0)),
            pl.BlockSpec((Ep, Hp), const),
            pl.BlockSpec((Hp, Hp), const),
            pl.BlockSpec((1, Hp), const),
            pl.BlockSpec((Hp, Cp), const),
            pl.BlockSpec((1, Cp), const),
        ],
        out_specs=pl.BlockSpec((Bt, Cp), lambda i: (i, 0)),
        scratch_shapes=[pltpu.VMEM((T, Bt, Hp), cdt)],
        compiler_params=pltpu.CompilerParams(
            dimension_semantics=("parallel",),
        ),
    )(x, w_ih_c, w_hh_c, b_rnn, w_fc_c, b_fc_p)

    if (Bp, Cp) != (B, C):
        out_padded = out_padded[:B, :C]
    return out_padded
